# baseline (speedup 1.0000x reference)
import functools

import jax
import jax.numpy as jnp
from jax import lax
from jax.experimental import pallas as pl
from jax.experimental.pallas import tpu as pltpu
from jax.experimental.pallas import tpu_sc as plsc

_TOKENS = 32768
_D_IN = 768
_D_HID = 256
_N_EXPERTS = 64
_BLOCK = 4096

_NW = 32
_TPW = _TOKENS // _NW
_LANES = 16
_CHUNKS = _TPW // _LANES


def _mlp_body(x_ref, w1_ref, b1_ref, w2_ref, b2_ref, logits_ref):
    h = jnp.dot(x_ref[...], w1_ref[...], preferred_element_type=jnp.float32)
    h = jnp.maximum(h + b1_ref[...], 0.0)
    logits = jnp.dot(h, w2_ref[...], preferred_element_type=jnp.float32)
    logits_ref[...] = logits + b2_ref[...]


def _sc_topk_body(logits_hbm, i1_hbm, i2_hbm, g1_hbm, g2_hbm,
                  lv, i1v, i2v, g1v, g2v):
    wid = lax.axis_index("s") * 2 + lax.axis_index("c")
    base = wid * _TPW
    lvf = lv
    pltpu.sync_copy(logits_hbm.at[pl.ds(base * _N_EXPERTS,
                                        _TPW * _N_EXPERTS)], lvf)

    lane = lax.broadcasted_iota(jnp.int32, (_LANES,), 0)

    def chunk(c, _):
        flat0 = (c * _LANES + lane) * _N_EXPERTS
        m1 = plsc.load_gather(lvf, [flat0])
        i1 = jnp.zeros((_LANES,), jnp.int32)
        m2 = jnp.full((_LANES,), -jnp.inf, jnp.float32)
        i2 = jnp.zeros((_LANES,), jnp.int32)
        for e in range(1, _N_EXPERTS):
            v = plsc.load_gather(lvf, [flat0 + e])
            gt1 = v > m1
            gt2 = v > m2
            m2 = jnp.where(gt1, m1, jnp.where(gt2, v, m2))
            i2 = jnp.where(gt1, i1, jnp.where(gt2, e, i2))
            m1 = jnp.where(gt1, v, m1)
            i1 = jnp.where(gt1, e, i1)
        ex = jnp.exp(m2 - m1)
        denom = 1.0 + ex
        sl = pl.ds(c * _LANES, _LANES)
        i1v[sl] = i1
        i2v[sl] = i2
        g1v[sl] = 1.0 / denom
        g2v[sl] = ex / denom
        return _

    lax.fori_loop(0, _CHUNKS, chunk, 0)
    pltpu.sync_copy(i1v, i1_hbm.at[pl.ds(base, _TPW)])
    pltpu.sync_copy(i2v, i2_hbm.at[pl.ds(base, _TPW)])
    pltpu.sync_copy(g1v, g1_hbm.at[pl.ds(base, _TPW)])
    pltpu.sync_copy(g2v, g2_hbm.at[pl.ds(base, _TPW)])


_sc_topk = functools.partial(
    pl.kernel,
    mesh=plsc.VectorSubcoreMesh(core_axis_name="c", subcore_axis_name="s"),
    out_type=[
        jax.ShapeDtypeStruct((_TOKENS,), jnp.int32),
        jax.ShapeDtypeStruct((_TOKENS,), jnp.int32),
        jax.ShapeDtypeStruct((_TOKENS,), jnp.float32),
        jax.ShapeDtypeStruct((_TOKENS,), jnp.float32),
    ],
    scratch_types=[
        pltpu.VMEM((_TPW * _N_EXPERTS,), jnp.float32),
        pltpu.VMEM((_TPW,), jnp.int32),
        pltpu.VMEM((_TPW,), jnp.int32),
        pltpu.VMEM((_TPW,), jnp.float32),
        pltpu.VMEM((_TPW,), jnp.float32),
    ],
    compiler_params=pltpu.CompilerParams(needs_layout_passes=False),
)(_sc_topk_body)


@jax.jit
def kernel(x, W1, b1, W2, b2):
    b1r = b1.reshape(1, _D_HID)
    b2r = b2.reshape(1, _N_EXPERTS)
    grid = (_TOKENS // _BLOCK,)
    logits = pl.pallas_call(
        _mlp_body,
        grid=grid,
        in_specs=[
            pl.BlockSpec((_BLOCK, _D_IN), lambda i: (i, 0)),
            pl.BlockSpec((_D_IN, _D_HID), lambda i: (0, 0)),
            pl.BlockSpec((1, _D_HID), lambda i: (0, 0)),
            pl.BlockSpec((_D_HID, _N_EXPERTS), lambda i: (0, 0)),
            pl.BlockSpec((1, _N_EXPERTS), lambda i: (0, 0)),
        ],
        out_specs=pl.BlockSpec((_BLOCK, _N_EXPERTS), lambda i: (i, 0)),
        out_shape=jax.ShapeDtypeStruct((_TOKENS, _N_EXPERTS), jnp.float32),
        compiler_params=pltpu.CompilerParams(
            dimension_semantics=("parallel",)),
    )(x, W1, b1r, W2, b2r)
    i1, i2, g1, g2 = _sc_topk(logits.reshape(_TOKENS * _N_EXPERTS))
    idx = jnp.stack([i1, i2], axis=-1)
    gates = jnp.stack([g1, g2], axis=-1)
    return idx, gates
